# baseline (speedup 1.0000x reference)
import functools

import jax
import jax.numpy as jnp
from jax import lax
from jax.experimental import pallas as pl
from jax.experimental.pallas import tpu as pltpu
from jax.experimental.pallas import tpu_sc as plsc

NUM_ENTRIES = 100000
LEARNABLE_SIZE = 4
D = 768
BATCH = 4096
ROW = LEARNABLE_SIZE * D

NC = 2
NS = 16
NW = NC * NS
B_PER_W = BATCH // NW
CHUNK = 8
NCHUNK = B_PER_W // CHUNK
NBUF = 5

_mesh = plsc.VectorSubcoreMesh(core_axis_name="c", subcore_axis_name="s")


@functools.partial(
    pl.kernel,
    mesh=_mesh,
    out_type=jax.ShapeDtypeStruct((BATCH, LEARNABLE_SIZE, D), jnp.float32),
    scratch_types=[
        pltpu.VMEM((B_PER_W,), jnp.int32),
        pltpu.VMEM((NBUF, CHUNK, LEARNABLE_SIZE, D), jnp.float32),
    ]
    + [pltpu.SemaphoreType.DMA] * (2 * NBUF),
)
def _gather_kernel(idx_hbm, table_hbm, out_hbm, idx_v, rows_v, *sems):
    gsems = sems[:NBUF]
    osems = sems[NBUF:]
    wid = lax.axis_index("s") * NC + lax.axis_index("c")
    base = wid * B_PER_W
    pltpu.sync_copy(idx_hbm.at[pl.ds(base, B_PER_W)], idx_v)

    def gather(c, b):
        return pltpu.async_copy(
            table_hbm.at[idx_v.at[pl.ds(c * CHUNK, CHUNK)]], rows_v.at[b],
            gsems[b])

    def out_copy(c, b):
        return pltpu.make_async_copy(
            rows_v.at[b], out_hbm.at[pl.ds(base + c * CHUNK, CHUNK)],
            osems[b])

    LAG = NBUF - 2
    for b in range(NBUF):
        gather(b, b)
    for c in range(NCHUNK):
        b = c % NBUF
        pltpu.make_async_copy(
            table_hbm.at[idx_v.at[pl.ds(c * CHUNK, CHUNK)]], rows_v.at[b],
            gsems[b]).wait()
        out_copy(c, b).start()
        d = c - LAG
        m = d + NBUF
        if d >= 0 and m < NCHUNK:
            out_copy(d, d % NBUF).wait()
            gather(m, d % NBUF)
    for c in range(NCHUNK - NBUF, NCHUNK):
        out_copy(c, c % NBUF).wait()


def kernel(x, learnable_matrix):
    return _gather_kernel(x.astype(jnp.int32), learnable_matrix)
